# baseline (speedup 1.0000x reference)
import functools

import jax
import jax.numpy as jnp
from jax import lax
from jax.experimental import pallas as pl
from jax.experimental.pallas import tpu as pltpu
from jax.experimental.pallas import tpu_sc as plsc

N = 10000
D = 128
NPAD = 10240
NC = 2
NS = 16
NW = NC * NS
CHUNK = 128
DEGW = 16
BN = 512
_MESH = plsc.VectorSubcoreMesh(core_axis_name="c", subcore_axis_name="s")
_RPT = NPAD // NS


def _sc_deg_body(row_hbm, ones_hbm, z_hbm, out_hbm, row_v, ones_v, acc_sh, nch):
    c = lax.axis_index("c")
    s = lax.axis_index("s")
    t = c * NS + s
    r0 = s * _RPT
    pltpu.sync_copy(z_hbm.at[pl.ds(r0, _RPT)], acc_sh.at[pl.ds(r0, _RPT)])
    pltpu.sync_copy(row_hbm.at[t], row_v)
    pltpu.sync_copy(ones_hbm, ones_v)
    plsc.subcore_barrier()

    def body(j, carry):
        pltpu.sync_copy(ones_v, acc_sh.at[row_v.at[j]], add=True)
        return carry

    lax.fori_loop(0, nch, body, 0)
    plsc.subcore_barrier()
    pltpu.sync_copy(acc_sh.at[pl.ds(r0, _RPT)], out_hbm.at[c].at[pl.ds(r0, _RPT)])


def _sc_deg(row3, ones, zdeg):
    nch = row3.shape[1]
    body = functools.partial(_sc_deg_body, nch=nch)
    return pl.kernel(
        body,
        out_type=jax.ShapeDtypeStruct((NC, NPAD), jnp.float32),
        mesh=_MESH,
        scratch_types=[
            pltpu.VMEM((nch, CHUNK), jnp.int32),
            pltpu.VMEM((CHUNK,), jnp.float32),
            pltpu.VMEM_SHARED((NPAD,), jnp.float32),
        ],
    )(row3, ones, zdeg)


FRACA = 0.53


def _sc_spmv_body(u_hbm, rowA, colA, rowB, colB, z_hbm, out_hbm,
                  row_v, col_v, rows_v, acc_sh, nchA, nchB):
    c = lax.axis_index("c")
    s = lax.axis_index("s")
    r0 = s * _RPT
    pltpu.sync_copy(z_hbm.at[pl.ds(r0, _RPT)], acc_sh.at[pl.ds(r0, _RPT)])

    @pl.when(c == 0)
    def _():
        pltpu.sync_copy(rowA.at[s], row_v.at[pl.ds(0, nchA)])
        pltpu.sync_copy(colA.at[s], col_v.at[pl.ds(0, nchA)])

    @pl.when(c == 1)
    def _():
        pltpu.sync_copy(rowB.at[s], row_v.at[pl.ds(0, nchB)])
        pltpu.sync_copy(colB.at[s], col_v.at[pl.ds(0, nchB)])

    plsc.subcore_barrier()

    def step(j, carry):
        pltpu.sync_copy(u_hbm.at[row_v.at[j]], rows_v)
        pltpu.sync_copy(rows_v, acc_sh.at[col_v.at[j]], add=True)
        return carry

    nch = jnp.where(c == 0, nchA, nchB)
    lax.fori_loop(0, nch, step, 0)
    plsc.subcore_barrier()
    pltpu.sync_copy(acc_sh.at[pl.ds(r0, _RPT)], out_hbm.at[c].at[pl.ds(r0, _RPT)])


def _sc_spmv(u_pad, rowA, colA, rowB, colB, zsp):
    nchA, nchB = rowA.shape[1], rowB.shape[1]
    nchmax = max(nchA, nchB)
    body = functools.partial(_sc_spmv_body, nchA=nchA, nchB=nchB)
    return pl.kernel(
        body,
        out_type=jax.ShapeDtypeStruct((NC, NPAD, D), jnp.float32),
        mesh=_MESH,
        scratch_types=[
            pltpu.VMEM((nchmax, CHUNK), jnp.int32),
            pltpu.VMEM((nchmax, CHUNK), jnp.int32),
            pltpu.VMEM((CHUNK, D), jnp.float32),
            pltpu.VMEM_SHARED((NPAD, D), jnp.float32),
        ],
    )(u_pad, rowA, colA, rowB, colB, zsp)


def _dinv_of(deg_ref):
    degs = deg_ref[0, :] + deg_ref[1, :]
    return jnp.where(degs > 0, lax.rsqrt(degs), 0.0)


def _scale1_body(x_ref, deg_ref, o_ref):
    dinv = _dinv_of(deg_ref)
    o_ref[...] = dinv[:, None] * x_ref[...]


def _scale2_body(g_ref, deg_ref, o_ref):
    dinv = _dinv_of(deg_ref)
    o_ref[...] = (dinv * dinv)[:, None] * (g_ref[0] + g_ref[1])


def _combine_body(src_ref, g1_ref, g2_ref, deg_ref, w_ref, b_ref, *outs,
                  relu, emit_u):
    dinv = _dinv_of(deg_ref)
    g1 = dinv[:, None] * (g1_ref[0] + g1_ref[1])
    g2 = dinv[:, None] * (g2_ref[0] + g2_ref[1])
    w02 = w_ref[0] - w_ref[2]
    acc = jnp.dot(src_ref[...], w02, preferred_element_type=jnp.float32)
    acc = acc - jnp.dot(g1, w_ref[1], preferred_element_type=jnp.float32)
    acc = acc + 2.0 * jnp.dot(g2, w_ref[2], preferred_element_type=jnp.float32)
    acc = acc + b_ref[...]
    if relu:
        acc = jnp.maximum(acc, 0.0)
        i = pl.program_id(0)
        rows = i * BN + lax.broadcasted_iota(jnp.int32, (BN, 1), 0)
        acc = jnp.where(rows < N, acc, 0.0)
    outs[0][...] = acc
    if emit_u:
        outs[1][...] = dinv[:, None] * acc


_ROWS_SPEC = pl.BlockSpec((BN, D), lambda i: (i, 0))
_PART_SPEC = pl.BlockSpec((NC, BN, D), lambda i: (0, i, 0))
_DEG_SPEC = pl.BlockSpec((NC, BN), lambda i: (0, i))
_GRID = (NPAD // BN,)
_ROWS_TY = jax.ShapeDtypeStruct((NPAD, D), jnp.float32)


def _tc_scale1(x_pad, deg2):
    return pl.pallas_call(
        _scale1_body, grid=_GRID,
        in_specs=[_ROWS_SPEC, _DEG_SPEC],
        out_specs=_ROWS_SPEC, out_shape=_ROWS_TY,
    )(x_pad, deg2)


def _tc_scale2(g, deg2):
    return pl.pallas_call(
        _scale2_body, grid=_GRID,
        in_specs=[_PART_SPEC, _DEG_SPEC],
        out_specs=_ROWS_SPEC, out_shape=_ROWS_TY,
    )(g, deg2)


def _tc_combine(src, g1, g2, deg2, w, b, relu, emit_u):
    body = functools.partial(_combine_body, relu=relu, emit_u=emit_u)
    nout = 2 if emit_u else 1
    out = pl.pallas_call(
        body, grid=_GRID,
        in_specs=[_ROWS_SPEC, _PART_SPEC, _PART_SPEC, _DEG_SPEC,
                  pl.BlockSpec((3, D, D), lambda i: (0, 0, 0)),
                  pl.BlockSpec((1, D), lambda i: (0, 0))],
        out_specs=[_ROWS_SPEC] * nout,
        out_shape=[_ROWS_TY] * nout,
    )(src, g1, g2, deg2, w, b)
    return out


def kernel(x, edge_index, W1, b1, W2, b2):
    e = edge_index.shape[1]
    row = edge_index[0].astype(jnp.int32)
    col = edge_index[1].astype(jnp.int32)
    tot = -(-e // CHUNK)
    nchA = max(1, round(tot * FRACA / NS))
    nchB = -(-(tot - NS * nchA) // NS)
    epad = NS * (nchA + nchB) * CHUNK
    padv = jnp.full((epad - e,), N, dtype=jnp.int32)
    rowp = jnp.concatenate([row, padv])
    colp = jnp.concatenate([col, padv])
    ea = NS * nchA * CHUNK
    rowA = rowp[:ea].reshape(NS, nchA, CHUNK)
    colA = colp[:ea].reshape(NS, nchA, CHUNK)
    rowB = rowp[ea:].reshape(NS, nchB, CHUNK)
    colB = colp[ea:].reshape(NS, nchB, CHUNK)
    nchd = -(-e // (NW * CHUNK))
    padd = jnp.full((NW * nchd * CHUNK - e,), N, dtype=jnp.int32)
    row3d = jnp.concatenate([row, padd]).reshape(NW, nchd, CHUNK)
    x_pad = jnp.pad(x, ((0, NPAD - N), (0, 0)))
    zsp = jnp.zeros((NPAD, D), jnp.float32)
    zdeg = jnp.zeros((NPAD,), jnp.float32)
    ones = jnp.ones((CHUNK,), jnp.float32)

    deg2 = _sc_deg(row3d, ones, zdeg)
    u = _tc_scale1(x_pad, deg2)
    g1 = _sc_spmv(u, rowA, colA, rowB, colB, zsp)
    u2 = _tc_scale2(g1, deg2)
    g2 = _sc_spmv(u2, rowA, colA, rowB, colB, zsp)
    h, u = _tc_combine(x_pad, g1, g2, deg2, W1, b1.reshape(1, D),
                       relu=True, emit_u=True)
    g1b = _sc_spmv(u, rowA, colA, rowB, colB, zsp)
    u2b = _tc_scale2(g1b, deg2)
    g2b = _sc_spmv(u2b, rowA, colA, rowB, colB, zsp)
    out = _tc_combine(h, g1b, g2b, deg2, W2, b2.reshape(1, D),
                      relu=False, emit_u=False)
    return out[0][:N]


# speedup vs baseline: 1.0557x; 1.0557x over previous
import functools

import jax
import jax.numpy as jnp
from jax import lax
from jax.experimental import pallas as pl
from jax.experimental.pallas import tpu as pltpu
from jax.experimental.pallas import tpu_sc as plsc

N = 10000
D = 128
NPAD = 10240
NC = 2
NS = 16
NW = NC * NS
CHUNK = 128
DEGW = 16
BN = 512
_MESH = plsc.VectorSubcoreMesh(core_axis_name="c", subcore_axis_name="s")
_RPT = NPAD // NS


def _sc_deg_body(row_hbm, ones_hbm, z_hbm, out_hbm, row_v, ones_v, acc_sh, nch):
    c = lax.axis_index("c")
    s = lax.axis_index("s")
    t = c * NS + s
    r0 = s * _RPT
    pltpu.sync_copy(z_hbm.at[pl.ds(r0, _RPT)], acc_sh.at[pl.ds(r0, _RPT)])
    pltpu.sync_copy(row_hbm.at[t], row_v)
    pltpu.sync_copy(ones_hbm, ones_v)
    plsc.subcore_barrier()

    def body(j, carry):
        pltpu.sync_copy(ones_v, acc_sh.at[row_v.at[j]], add=True)
        return carry

    lax.fori_loop(0, nch, body, 0)
    plsc.subcore_barrier()
    pltpu.sync_copy(acc_sh.at[pl.ds(r0, _RPT)], out_hbm.at[c].at[pl.ds(r0, _RPT)])


def _sc_deg(row3, ones, zdeg):
    nch = row3.shape[1]
    body = functools.partial(_sc_deg_body, nch=nch)
    return pl.kernel(
        body,
        out_type=jax.ShapeDtypeStruct((NC, NPAD), jnp.float32),
        mesh=_MESH,
        scratch_types=[
            pltpu.VMEM((nch, CHUNK), jnp.int32),
            pltpu.VMEM((CHUNK,), jnp.float32),
            pltpu.VMEM_SHARED((NPAD,), jnp.float32),
        ],
    )(row3, ones, zdeg)


FRACA = 0.51


def _sc_spmv_body(u_hbm, rowA, colA, rowB, colB, z_hbm, out_hbm,
                  row_v, col_v, rows_v, acc_sh, nchA, nchB):
    c = lax.axis_index("c")
    s = lax.axis_index("s")
    r0 = s * _RPT
    pltpu.sync_copy(z_hbm.at[pl.ds(r0, _RPT)], acc_sh.at[pl.ds(r0, _RPT)])

    @pl.when(c == 0)
    def _():
        pltpu.sync_copy(rowA.at[s], row_v.at[pl.ds(0, nchA)])
        pltpu.sync_copy(colA.at[s], col_v.at[pl.ds(0, nchA)])

    @pl.when(c == 1)
    def _():
        pltpu.sync_copy(rowB.at[s], row_v.at[pl.ds(0, nchB)])
        pltpu.sync_copy(colB.at[s], col_v.at[pl.ds(0, nchB)])

    plsc.subcore_barrier()

    def step(j, carry):
        pltpu.sync_copy(u_hbm.at[row_v.at[j]], rows_v)
        pltpu.sync_copy(rows_v, acc_sh.at[col_v.at[j]], add=True)
        return carry

    nch = jnp.where(c == 0, nchA, nchB)
    lax.fori_loop(0, nch, step, 0)
    plsc.subcore_barrier()
    pltpu.sync_copy(acc_sh.at[pl.ds(r0, _RPT)], out_hbm.at[c].at[pl.ds(r0, _RPT)])


def _sc_spmv(u_pad, rowA, colA, rowB, colB, zsp):
    nchA, nchB = rowA.shape[1], rowB.shape[1]
    nchmax = max(nchA, nchB)
    body = functools.partial(_sc_spmv_body, nchA=nchA, nchB=nchB)
    return pl.kernel(
        body,
        out_type=jax.ShapeDtypeStruct((NC, NPAD, D), jnp.float32),
        mesh=_MESH,
        scratch_types=[
            pltpu.VMEM((nchmax, CHUNK), jnp.int32),
            pltpu.VMEM((nchmax, CHUNK), jnp.int32),
            pltpu.VMEM((CHUNK, D), jnp.float32),
            pltpu.VMEM_SHARED((NPAD, D), jnp.float32),
        ],
    )(u_pad, rowA, colA, rowB, colB, zsp)


def _dinv_of(deg_ref):
    degs = deg_ref[0, :] + deg_ref[1, :]
    return jnp.where(degs > 0, lax.rsqrt(degs), 0.0)


def _scale1_body(x_ref, deg_ref, o_ref):
    dinv = _dinv_of(deg_ref)
    o_ref[...] = dinv[:, None] * x_ref[...]


def _scale2_body(g_ref, deg_ref, o_ref):
    dinv = _dinv_of(deg_ref)
    o_ref[...] = (dinv * dinv)[:, None] * (g_ref[0] + g_ref[1])


def _combine_body(src_ref, g1_ref, g2_ref, deg_ref, w_ref, b_ref, *outs,
                  relu, emit_u):
    dinv = _dinv_of(deg_ref)
    g1 = dinv[:, None] * (g1_ref[0] + g1_ref[1])
    g2 = dinv[:, None] * (g2_ref[0] + g2_ref[1])
    w02 = w_ref[0] - w_ref[2]
    acc = jnp.dot(src_ref[...], w02, preferred_element_type=jnp.float32)
    acc = acc - jnp.dot(g1, w_ref[1], preferred_element_type=jnp.float32)
    acc = acc + 2.0 * jnp.dot(g2, w_ref[2], preferred_element_type=jnp.float32)
    acc = acc + b_ref[...]
    if relu:
        acc = jnp.maximum(acc, 0.0)
        i = pl.program_id(0)
        rows = i * BN + lax.broadcasted_iota(jnp.int32, (BN, 1), 0)
        acc = jnp.where(rows < N, acc, 0.0)
    outs[0][...] = acc
    if emit_u:
        outs[1][...] = dinv[:, None] * acc


_ROWS_SPEC = pl.BlockSpec((BN, D), lambda i: (i, 0))
_PART_SPEC = pl.BlockSpec((NC, BN, D), lambda i: (0, i, 0))
_DEG_SPEC = pl.BlockSpec((NC, BN), lambda i: (0, i))
_GRID = (NPAD // BN,)
_ROWS_TY = jax.ShapeDtypeStruct((NPAD, D), jnp.float32)


def _tc_scale1(x_pad, deg2):
    return pl.pallas_call(
        _scale1_body, grid=_GRID,
        in_specs=[_ROWS_SPEC, _DEG_SPEC],
        out_specs=_ROWS_SPEC, out_shape=_ROWS_TY,
    )(x_pad, deg2)


def _tc_scale2(g, deg2):
    return pl.pallas_call(
        _scale2_body, grid=_GRID,
        in_specs=[_PART_SPEC, _DEG_SPEC],
        out_specs=_ROWS_SPEC, out_shape=_ROWS_TY,
    )(g, deg2)


def _tc_combine(src, g1, g2, deg2, w, b, relu, emit_u):
    body = functools.partial(_combine_body, relu=relu, emit_u=emit_u)
    nout = 2 if emit_u else 1
    out = pl.pallas_call(
        body, grid=_GRID,
        in_specs=[_ROWS_SPEC, _PART_SPEC, _PART_SPEC, _DEG_SPEC,
                  pl.BlockSpec((3, D, D), lambda i: (0, 0, 0)),
                  pl.BlockSpec((1, D), lambda i: (0, 0))],
        out_specs=[_ROWS_SPEC] * nout,
        out_shape=[_ROWS_TY] * nout,
    )(src, g1, g2, deg2, w, b)
    return out


def kernel(x, edge_index, W1, b1, W2, b2):
    e = edge_index.shape[1]
    row = edge_index[0].astype(jnp.int32)
    col = edge_index[1].astype(jnp.int32)
    tot = -(-e // CHUNK)
    nchA = max(1, round(tot * FRACA / NS))
    nchB = -(-(tot - NS * nchA) // NS)
    epad = NS * (nchA + nchB) * CHUNK
    padv = jnp.full((epad - e,), N, dtype=jnp.int32)
    rowp = jnp.concatenate([row, padv])
    colp = jnp.concatenate([col, padv])
    ea = NS * nchA * CHUNK
    rowA = rowp[:ea].reshape(NS, nchA, CHUNK)
    colA = colp[:ea].reshape(NS, nchA, CHUNK)
    rowB = rowp[ea:].reshape(NS, nchB, CHUNK)
    colB = colp[ea:].reshape(NS, nchB, CHUNK)
    nchd = -(-e // (NW * CHUNK))
    padd = jnp.full((NW * nchd * CHUNK - e,), N, dtype=jnp.int32)
    row3d = jnp.concatenate([row, padd]).reshape(NW, nchd, CHUNK)
    x_pad = jnp.pad(x, ((0, NPAD - N), (0, 0)))
    zsp = jnp.zeros((NPAD, D), jnp.float32)
    zdeg = jnp.zeros((NPAD,), jnp.float32)
    ones = jnp.ones((CHUNK,), jnp.float32)

    deg2 = _sc_deg(row3d, ones, zdeg)
    u = _tc_scale1(x_pad, deg2)
    g1 = _sc_spmv(u, rowA, colA, rowB, colB, zsp)
    u2 = _tc_scale2(g1, deg2)
    g2 = _sc_spmv(u2, rowA, colA, rowB, colB, zsp)
    h, u = _tc_combine(x_pad, g1, g2, deg2, W1, b1.reshape(1, D),
                       relu=True, emit_u=True)
    g1b = _sc_spmv(u, rowA, colA, rowB, colB, zsp)
    u2b = _tc_scale2(g1b, deg2)
    g2b = _sc_spmv(u2b, rowA, colA, rowB, colB, zsp)
    out = _tc_combine(h, g1b, g2b, deg2, W2, b2.reshape(1, D),
                      relu=False, emit_u=False)
    return out[0][:N]
